# Initial kernel scaffold; baseline (speedup 1.0000x reference)
import jax
import jax.numpy as jnp
from jax.experimental import pallas as pl


def kernel(sequence, wte):
    raise NotImplementedError("write your pallas kernel here")


# speedup vs baseline: 1.4780x; 1.4780x over previous
import functools

import jax
import jax.numpy as jnp
from jax import lax
from jax.experimental import pallas as pl
from jax.experimental.pallas import tpu as pltpu
from jax.experimental.pallas import tpu_sc as plsc

_D = 768
_NC = 2
_NS = 16
_NW = _NC * _NS
_B = 8192
_BPW = _B // _NW
_CH = 64
_NCH = _BPW // _CH


def _sc_embedding_lookup(idx_flat, wte):
    mesh = plsc.VectorSubcoreMesh(core_axis_name="c", subcore_axis_name="s")

    @functools.partial(
        pl.kernel,
        mesh=mesh,
        out_type=jax.ShapeDtypeStruct((_B, _D), jnp.float32),
        scratch_types=[
            pltpu.VMEM((_BPW,), jnp.int32),
            pltpu.VMEM((_CH, _D), jnp.float32),
            pltpu.VMEM((_CH, _D), jnp.float32),
            pltpu.SemaphoreType.DMA,
            pltpu.SemaphoreType.DMA,
            pltpu.SemaphoreType.DMA,
            pltpu.SemaphoreType.DMA,
        ],
    )
    def body(idx_hbm, table_hbm, out_hbm, idx_v, rows0, rows1,
             gsem0, gsem1, ssem0, ssem1):
        wid = lax.axis_index("s") * _NC + lax.axis_index("c")
        base = wid * _BPW
        pltpu.sync_copy(idx_hbm.at[pl.ds(base, _BPW)], idx_v)

        rows = (rows0, rows1)
        gsems = (gsem0, gsem1)
        ssems = (ssem0, ssem1)

        pltpu.async_copy(
            table_hbm.at[idx_v.at[pl.ds(0, _CH)]], rows0, gsem0)

        for c in range(_NCH):
            p = c % 2
            q = (c + 1) % 2
            if c + 1 < _NCH:
                if c + 1 >= 2:
                    pltpu.make_async_copy(
                        rows[q], out_hbm.at[pl.ds(base, _CH)], ssems[q]).wait()
                pltpu.async_copy(
                    table_hbm.at[idx_v.at[pl.ds((c + 1) * _CH, _CH)]],
                    rows[q], gsems[q])
            pltpu.make_async_copy(
                table_hbm.at[idx_v.at[pl.ds(c * _CH, _CH)]],
                rows[p], gsems[p]).wait()
            pltpu.async_copy(
                rows[p], out_hbm.at[pl.ds(base + c * _CH, _CH)], ssems[p])

        pltpu.make_async_copy(
            rows[(_NCH - 2) % 2], out_hbm.at[pl.ds(base, _CH)],
            ssems[(_NCH - 2) % 2]).wait()
        pltpu.make_async_copy(
            rows[(_NCH - 1) % 2], out_hbm.at[pl.ds(base, _CH)],
            ssems[(_NCH - 1) % 2]).wait()

    return body(idx_flat, wte)


def kernel(sequence, wte):
    b, s = sequence.shape
    idx_flat = sequence.reshape(b * s).astype(jnp.int32)
    out = _sc_embedding_lookup(idx_flat, wte)
    return out.reshape(b, s, _D)
